# Initial kernel scaffold; baseline (speedup 1.0000x reference)
import jax
import jax.numpy as jnp
from jax.experimental import pallas as pl


def kernel(x, edge_index, edge_type, W, b):
    raise NotImplementedError("write your pallas kernel here")


# speedup vs baseline: 4.5588x; 4.5588x over previous
import functools

import jax
import jax.numpy as jnp
from jax import lax
from jax.experimental import pallas as pl
from jax.experimental.pallas import tpu as pltpu
from jax.experimental.pallas import tpu_sc as plsc

N = 10000
E = 320000
T = 7
C = 128
AVG = 7

NB = 10
BN = N // NB

NC, NS, L = 2, 16, 16
NW = NC * NS
EPW = E // NW
K = 80
NCHUNK = EPW // K
ZR = 80
NZCH = N // ZR


def _mm_body(x_ref, w_ref, y_ref):
    y_ref[...] = jnp.dot(x_ref[...], w_ref[0],
                         preferred_element_type=jnp.float32)


def _compute_y(x, w3):
    return pl.pallas_call(
        _mm_body,
        grid=(NB, T),
        in_specs=[
            pl.BlockSpec((BN, C), lambda nb, t: (nb, 0)),
            pl.BlockSpec((1, C, C), lambda nb, t: (t, 0, 0)),
        ],
        out_specs=pl.BlockSpec((BN, C), lambda nb, t: (t * NB + nb, 0)),
        out_shape=jax.ShapeDtypeStruct((T * N, C), jnp.float32),
    )(x, w3)


def _sc_body(y_hbm, src_hbm, dst_hbm, et_hbm, part_hbm,
             src_v, dst_v, et_v, gidx_v, rows_v, zrow_v, acc_sh, sem):
    cid = lax.axis_index("c")
    sid = lax.axis_index("s")
    wid = sid * NC + cid
    base = wid * EPW

    def _zero_zbuf(i, _):
        for c8 in range(C // L):
            zrow_v[i, pl.ds(c8 * L, L)] = jnp.zeros((L,), jnp.float32)
        return 0
    lax.fori_loop(0, ZR, _zero_zbuf, 0)

    def _zero_acc(k, _):
        ch = sid + NS * k
        @pl.when(ch < NZCH)
        def _():
            pltpu.sync_copy(zrow_v, acc_sh.at[pl.ds(ch * ZR, ZR), :])
        return 0
    lax.fori_loop(0, (NZCH + NS - 1) // NS, _zero_acc, 0)
    plsc.subcore_barrier()

    def _chunk(j, _):
        off = base + j * K
        pltpu.sync_copy(src_hbm.at[pl.ds(off, K)], src_v)
        pltpu.sync_copy(dst_hbm.at[pl.ds(off, K)], dst_v)
        pltpu.sync_copy(et_hbm.at[pl.ds(off, K)], et_v)
        for i in range(K // L):
            s = pl.ds(i * L, L)
            gidx_v[s] = et_v[s] * N + src_v[s]
        pltpu.async_copy(y_hbm.at[gidx_v], rows_v, sem).wait()
        pltpu.sync_copy(rows_v, acc_sh.at[dst_v], add=True)
        return 0

    lax.fori_loop(0, NCHUNK, _chunk, 0)
    plsc.subcore_barrier()

    def _writeout(k, _):
        ch = sid + NS * k
        @pl.when(ch < NZCH)
        def _():
            pltpu.sync_copy(acc_sh.at[pl.ds(ch * ZR, ZR), :],
                            part_hbm.at[cid, pl.ds(ch * ZR, ZR), :])
        return 0
    lax.fori_loop(0, (NZCH + NS - 1) // NS, _writeout, 0)


def _aggregate(y, src, dst, et):
    mesh = plsc.VectorSubcoreMesh(core_axis_name="c", subcore_axis_name="s")
    f = pl.kernel(
        _sc_body,
        out_type=jax.ShapeDtypeStruct((NC, N, C), jnp.float32),
        mesh=mesh,
        scratch_types=[
            pltpu.VMEM((K,), jnp.int32),
            pltpu.VMEM((K,), jnp.int32),
            pltpu.VMEM((K,), jnp.int32),
            pltpu.VMEM((K,), jnp.int32),
            pltpu.VMEM((K, C), jnp.float32),
            pltpu.VMEM((ZR, C), jnp.float32),
            pltpu.VMEM_SHARED((N, C), jnp.float32),
            pltpu.SemaphoreType.DMA,
        ],
    )
    return f(y, src, dst, et)


def _ep_body(p_ref, b_ref, o_ref):
    o_ref[...] = (p_ref[0] + p_ref[1]) * jnp.float32(1.0 / AVG) + b_ref[...]


def _epilogue(part, b2):
    return pl.pallas_call(
        _ep_body,
        grid=(NB,),
        in_specs=[
            pl.BlockSpec((NC, BN, C), lambda nb: (0, nb, 0)),
            pl.BlockSpec((1, C), lambda nb: (0, 0)),
        ],
        out_specs=pl.BlockSpec((BN, C), lambda nb: (nb, 0)),
        out_shape=jax.ShapeDtypeStruct((N, C), jnp.float32),
    )(part, b2)


def kernel(x, edge_index, edge_type, W, b):
    w3 = W.reshape(T, C, C)
    y = _compute_y(x, w3)
    part = _aggregate(y, edge_index[0], edge_index[1], edge_type)
    return _epilogue(part, b.reshape(1, C))
